# Initial kernel scaffold; baseline (speedup 1.0000x reference)
import jax
import jax.numpy as jnp
from jax.experimental import pallas as pl


def kernel(x, ln_scale, ff_ln_scale, wq, wk, wv, wo, w_gate, w1, w2, w3):
    raise NotImplementedError("write your pallas kernel here")


# speedup vs baseline: 1.5538x; 1.5538x over previous
import functools

import jax
import jax.numpy as jnp
from jax.experimental import pallas as pl
from jax.experimental.pallas import tpu as pltpu

B, S, D = 1, 2048, 1024
H, KVH = 16, 8
DH = D // H
E, TOPK = 8, 2
HID = 2048
EPS = 1e-5
ROPE_BASE = 1000000.0
HALF = DH // 2

_INTERPRET = False

BS = 256
BQ = 256
HC = 2
HCHUNK = HID // HC
TCHUNK = 512


def _rms_norm(x, scale):
    var = jnp.mean(x * x, axis=-1, keepdims=True)
    return x * jax.lax.rsqrt(var + EPS) * scale


def _rope_rows(t, block_start):
    n = t.shape[1]
    pos = block_start + jax.lax.broadcasted_iota(
        jnp.int32, t.shape, 0).astype(jnp.float32)
    col = jax.lax.broadcasted_iota(jnp.int32, t.shape, 1)
    colmod = jax.lax.rem(col, DH)
    f = jax.lax.rem(colmod, HALF)
    inv_freq = jnp.exp(f.astype(jnp.float32) * (-jnp.log(ROPE_BASE) / HALF))
    ang = pos * inv_freq
    cos = jnp.cos(ang)
    sin = jnp.sin(ang)
    first_half = colmod < HALF
    shifted = jnp.where(first_half,
                        pltpu.roll(t, n - HALF, axis=1),
                        pltpu.roll(t, HALF, axis=1))
    sign = jnp.where(first_half, -1.0, 1.0)
    return t * cos + shifted * sin * sign


def _qkv_kernel(x_ref, lns_ref, wq_ref, wk_ref, wv_ref, q_ref, k_ref, v_ref):
    i = pl.program_id(0)
    h = _rms_norm(x_ref[...], lns_ref[...])
    q = jnp.dot(h, wq_ref[...], preferred_element_type=jnp.float32)
    k = jnp.dot(h, wk_ref[...], preferred_element_type=jnp.float32)
    v = jnp.dot(h, wv_ref[...], preferred_element_type=jnp.float32)
    start = (i * BS).astype(jnp.float32)
    q_ref[...] = _rope_rows(q, start)
    k_ref[...] = _rope_rows(k, start)
    v_ref[...] = v


def _attn_kernel(q_ref, k_ref, v_ref, o_ref):
    i = pl.program_id(1)
    q = q_ref[...] * (DH ** -0.5)
    row = i * BQ + jax.lax.broadcasted_iota(jnp.int32, (BQ, S), 0)
    colid = jax.lax.broadcasted_iota(jnp.int32, (BQ, S), 1)
    causal = colid <= row
    for hh in range(4):
        qh = q[:, hh * DH:(hh + 1) * DH]
        kv_lo = (hh // 2) * DH
        kh = k_ref[:, kv_lo:kv_lo + DH]
        vh = v_ref[:, kv_lo:kv_lo + DH]
        s = jax.lax.dot_general(qh, kh, (((1,), (1,)), ((), ())),
                                preferred_element_type=jnp.float32)
        s = jnp.where(causal, s, -1e9)
        m = jnp.max(s, axis=1, keepdims=True)
        p = jnp.exp(s - m)
        l = jnp.sum(p, axis=1, keepdims=True)
        o = jnp.dot(p, vh, preferred_element_type=jnp.float32)
        o_ref[:, hh * DH:(hh + 1) * DH] = o / l


def _post_attn_kernel(x_ref, attn_ref, wo_ref, ffs_ref, wg_ref,
                      x2_ref, h2_ref, gates_ref):
    x2 = x_ref[...] + jnp.dot(attn_ref[...], wo_ref[...],
                              preferred_element_type=jnp.float32)
    x2_ref[...] = x2
    h2 = _rms_norm(x2, ffs_ref[...])
    h2_ref[...] = h2
    logits = jnp.dot(h2, wg_ref[...], preferred_element_type=jnp.float32)
    p = jax.nn.softmax(logits, axis=1)
    eidx = jax.lax.broadcasted_iota(jnp.int32, p.shape, 1)
    v1 = jnp.max(p, axis=1, keepdims=True)
    i1 = jnp.min(jnp.where(p == v1, eidx, E), axis=1, keepdims=True)
    p2 = jnp.where(eidx == i1, -1.0, p)
    v2 = jnp.max(p2, axis=1, keepdims=True)
    i2 = jnp.min(jnp.where(p2 == v2, eidx, E), axis=1, keepdims=True)
    vsum = v1 + v2
    gates_ref[...] = (jnp.where(eidx == i1, v1 / vsum, 0.0)
                      + jnp.where(eidx == i2, v2 / vsum, 0.0))


def _moe_dense_kernel(h2_ref, x2_ref, gates_ref, w1_ref, w3_ref, w2_ref,
                      out_ref):
    e = pl.program_id(0)
    c = pl.program_id(1)
    first = (e == 0) & (c == 0)
    eidx = jax.lax.broadcasted_iota(jnp.int32, (TCHUNK, E), 1)
    w1 = w1_ref[0]
    w3 = w3_ref[0]
    w2 = w2_ref[0]
    for tc in range(S // TCHUNK):
        sl = pl.ds(tc * TCHUNK, TCHUNK)
        t = h2_ref[sl, :]
        g = jnp.sum(jnp.where(eidx == e, gates_ref[sl, :], 0.0),
                    axis=1, keepdims=True)
        h1 = jnp.dot(t, w1, preferred_element_type=jnp.float32)
        h3 = jnp.dot(t, w3, preferred_element_type=jnp.float32)
        act = (h1 * jax.nn.sigmoid(h1)) * h3
        contrib = jnp.dot(act, w2, preferred_element_type=jnp.float32) * g

        @pl.when(first)
        def _():
            out_ref[sl, :] = x2_ref[sl, :] + contrib

        @pl.when(jnp.logical_not(first))
        def _():
            out_ref[sl, :] = out_ref[sl, :] + contrib


def kernel(x, ln_scale, ff_ln_scale, wq, wk, wv, wo, w_gate, w1, w2, w3):
    xs = x.reshape(S, D)
    lns = ln_scale.reshape(1, D)
    ffs = ff_ln_scale.reshape(1, D)

    q, k, v = pl.pallas_call(
        _qkv_kernel,
        grid=(S // BS,),
        in_specs=[
            pl.BlockSpec((BS, D), lambda i: (i, 0)),
            pl.BlockSpec((1, D), lambda i: (0, 0)),
            pl.BlockSpec((D, H * DH), lambda i: (0, 0)),
            pl.BlockSpec((D, KVH * DH), lambda i: (0, 0)),
            pl.BlockSpec((D, KVH * DH), lambda i: (0, 0)),
        ],
        out_specs=[
            pl.BlockSpec((BS, H * DH), lambda i: (i, 0)),
            pl.BlockSpec((BS, KVH * DH), lambda i: (i, 0)),
            pl.BlockSpec((BS, KVH * DH), lambda i: (i, 0)),
        ],
        out_shape=[
            jax.ShapeDtypeStruct((S, H * DH), jnp.float32),
            jax.ShapeDtypeStruct((S, KVH * DH), jnp.float32),
            jax.ShapeDtypeStruct((S, KVH * DH), jnp.float32),
        ],
        interpret=_INTERPRET,
    )(xs, lns, wq, wk, wv)

    attn = pl.pallas_call(
        _attn_kernel,
        grid=(H // 4, S // BQ),
        in_specs=[
            pl.BlockSpec((BQ, 4 * DH), lambda g, i: (i, g)),
            pl.BlockSpec((S, 2 * DH), lambda g, i: (0, g)),
            pl.BlockSpec((S, 2 * DH), lambda g, i: (0, g)),
        ],
        out_specs=pl.BlockSpec((BQ, 4 * DH), lambda g, i: (i, g)),
        out_shape=jax.ShapeDtypeStruct((S, H * DH), jnp.float32),
        interpret=_INTERPRET,
    )(q, k, v)

    x2, h2, gates = pl.pallas_call(
        _post_attn_kernel,
        grid=(S // BS,),
        in_specs=[
            pl.BlockSpec((BS, D), lambda i: (i, 0)),
            pl.BlockSpec((BS, H * DH), lambda i: (i, 0)),
            pl.BlockSpec((H * DH, D), lambda i: (0, 0)),
            pl.BlockSpec((1, D), lambda i: (0, 0)),
            pl.BlockSpec((D, E), lambda i: (0, 0)),
        ],
        out_specs=[
            pl.BlockSpec((BS, D), lambda i: (i, 0)),
            pl.BlockSpec((BS, D), lambda i: (i, 0)),
            pl.BlockSpec((BS, E), lambda i: (i, 0)),
        ],
        out_shape=[
            jax.ShapeDtypeStruct((S, D), jnp.float32),
            jax.ShapeDtypeStruct((S, D), jnp.float32),
            jax.ShapeDtypeStruct((S, E), jnp.float32),
        ],
        interpret=_INTERPRET,
    )(xs, attn, wo, ffs, w_gate)

    out = pl.pallas_call(
        _moe_dense_kernel,
        grid=(E, HC),
        in_specs=[
            pl.BlockSpec((S, D), lambda e, c: (0, 0)),
            pl.BlockSpec((S, D), lambda e, c: (0, 0)),
            pl.BlockSpec((S, E), lambda e, c: (0, 0)),
            pl.BlockSpec((1, D, HCHUNK), lambda e, c: (e, 0, c)),
            pl.BlockSpec((1, D, HCHUNK), lambda e, c: (e, 0, c)),
            pl.BlockSpec((1, HCHUNK, D), lambda e, c: (e, c, 0)),
        ],
        out_specs=pl.BlockSpec((S, D), lambda e, c: (0, 0)),
        out_shape=jax.ShapeDtypeStruct((S, D), jnp.float32),
        interpret=_INTERPRET,
    )(h2, x2, gates, w1, w3, w2)

    return out.reshape(B, S, D)
